# baseline (speedup 1.0000x reference)
import functools
import math

import jax
import jax.numpy as jnp
from jax import lax
from jax.experimental import pallas as pl
from jax.experimental.pallas import tpu as pltpu
from jax.experimental.pallas import tpu_sc as plsc

NEG = -1e30
LANES = 16


def _scores_kernel(q_ref, k_ref, s_ref, *, sq, L, scale):
    sb = pl.program_id(1)
    q = q_ref[0, 0] * scale
    k = k_ref[0, 0]
    s = jax.lax.dot_general(q, k, (((1,), (1,)), ((), ())),
                            preferred_element_type=jnp.float32)
    row = jax.lax.broadcasted_iota(jnp.int32, (sq, L), 0) + sb * sq
    col = jax.lax.broadcasted_iota(jnp.int32, (sq, L), 1)
    s_ref[0, 0] = jnp.where(col <= row, s, NEG)


def _finish_kernel(s_ref, tm_ref, v_ref, o_ref):
    s = s_ref[0, 0]
    t = tm_ref[0, 0][:, 0:1]
    m = tm_ref[0, 0][:, 15:16]
    v = v_ref[0, 0]
    p = jnp.where(s >= t, jnp.exp(s - m), 0.0)
    z = jnp.sum(p, axis=1, keepdims=True)
    o = jax.lax.dot_general(p, v, (((1,), (0,)), ((), ())),
                            preferred_element_type=jnp.float32)
    o_ref[0, 0] = o / z


def _sc_topk(scores, nrows, L):
    info = plsc.get_sparse_core_info()
    nw = info.num_cores * info.num_subcores
    rows_per_w = nrows // nw
    chunk = 64
    n_chunks = rows_per_w // chunk
    mesh = plsc.VectorSubcoreMesh(core_axis_name="c", subcore_axis_name="s")

    @functools.partial(
        pl.kernel, mesh=mesh,
        compiler_params=pltpu.CompilerParams(needs_layout_passes=False),
        out_type=jax.ShapeDtypeStruct((nrows, LANES), jnp.float32),
        scratch_types=[
            pltpu.VMEM((chunk, L), jnp.float32),
            pltpu.VMEM((chunk, LANES), jnp.float32),
        ],
    )
    def topk_kernel(s_hbm, out_hbm, rows_v, top_v):
        wid = lax.axis_index("s") * info.num_cores + lax.axis_index("c")
        base = wid * rows_per_w

        def chunk_body(ci, _):
            start = base + ci * chunk
            pltpu.sync_copy(s_hbm.at[pl.ds(start, chunk)], rows_v)

            def row_body(r, __):
                merged = None
                for j in range(L // LANES):
                    vreg = jnp.sort(rows_v[r, pl.ds(j * LANES, LANES)])
                    if merged is None:
                        merged = vreg
                    else:
                        merged = jnp.sort(
                            jnp.maximum(merged, lax.rev(vreg, (0,))))
                top_v[r, :] = merged
                return __

            lax.fori_loop(0, chunk, row_body, 0)
            pltpu.sync_copy(top_v, out_hbm.at[pl.ds(start, chunk)])
            return _

        lax.fori_loop(0, n_chunks, chunk_body, 0)

    return topk_kernel(scores)


@jax.jit
def kernel(q, compressed_k, compressed_v):
    B, H, S, D = q.shape
    L = compressed_k.shape[2]
    SQ = 512
    grid = (H, S // SQ)
    scores = pl.pallas_call(
        functools.partial(_scores_kernel, sq=SQ, L=L,
                          scale=1.0 / math.sqrt(D)),
        grid=grid,
        in_specs=[
            pl.BlockSpec((1, 1, SQ, D), lambda h, sb: (0, h, sb, 0)),
            pl.BlockSpec((1, 1, L, D), lambda h, sb: (0, h, 0, 0)),
        ],
        out_specs=pl.BlockSpec((1, 1, SQ, L), lambda h, sb: (0, h, sb, 0)),
        out_shape=jax.ShapeDtypeStruct((B, H, S, L), jnp.float32),
    )(q, compressed_k)

    tm = _sc_topk(scores.reshape(H * S, L), H * S, L)
    tm4 = tm.reshape(B, H, S, LANES)

    out = pl.pallas_call(
        _finish_kernel,
        grid=grid,
        in_specs=[
            pl.BlockSpec((1, 1, SQ, L), lambda h, sb: (0, h, sb, 0)),
            pl.BlockSpec((1, 1, SQ, LANES), lambda h, sb: (0, h, sb, 0)),
            pl.BlockSpec((1, 1, L, D), lambda h, sb: (0, h, 0, 0)),
        ],
        out_specs=pl.BlockSpec((1, 1, SQ, D), lambda h, sb: (0, h, sb, 0)),
        out_shape=jax.ShapeDtypeStruct((B, H, S, D), jnp.float32),
    )(scores, tm4, compressed_v)
    return out


# speedup vs baseline: 2.6272x; 2.6272x over previous
import functools
import math

import jax
import jax.numpy as jnp
from jax.experimental import pallas as pl
from jax.experimental.pallas import tpu as pltpu

TOPK = 16
NEG = -1e30


def _attn_block_kernel(q_ref, k_ref, v_ref, o_ref, *, sq, L, scale):
    sb = pl.program_id(1)
    q = q_ref[0, 0] * scale
    k = k_ref[0, 0]
    v = v_ref[0, 0]

    s = jax.lax.dot_general(
        q, k, (((1,), (1,)), ((), ())),
        preferred_element_type=jnp.float32)

    row = jax.lax.broadcasted_iota(jnp.int32, (sq, L), 0) + sb * sq
    col = jax.lax.broadcasted_iota(jnp.int32, (sq, L), 1)
    s = jnp.where(col <= row, s, NEG)

    t_parts = []
    m_parts = []
    for r in range(2):
        sr = s[r * (sq // 2):(r + 1) * (sq // 2), :]
        a, b, c, d = (sr[:, 0:128], sr[:, 128:256],
                      sr[:, 256:384], sr[:, 384:512])
        hi1, lo1 = jnp.maximum(a, b), jnp.minimum(a, b)
        hi2, lo2 = jnp.maximum(c, d), jnp.minimum(c, d)
        w = jnp.maximum(hi1, hi2)
        th = jnp.minimum(hi1, hi2)
        tl = jnp.maximum(lo1, lo2)
        s4 = jnp.minimum(lo1, lo2)
        s2 = jnp.maximum(th, tl)
        s3 = jnp.minimum(th, tl)
        m = None
        t = None
        for p in range(TOPK):
            t = jnp.max(w, axis=1, keepdims=True)
            if p == 0:
                m = t
            if p < TOPK - 1:
                hit = w == t
                w = jnp.where(hit, s2, w)
                s2 = jnp.where(hit, s3, s2)
                s3 = jnp.where(hit, s4, s3)
                s4 = jnp.where(hit, NEG, s4)
        t_parts.append(t)
        m_parts.append(m)
    t = jnp.concatenate(t_parts, axis=0)
    m = jnp.concatenate(m_parts, axis=0)

    p_num = jnp.where(s >= t, jnp.exp(s - m), 0.0)
    z = jnp.sum(p_num, axis=1, keepdims=True)
    o = jax.lax.dot_general(
        p_num, v, (((1,), (0,)), ((), ())),
        preferred_element_type=jnp.float32)
    o_ref[0, 0] = o / z


@jax.jit
def kernel(q, compressed_k, compressed_v):
    B, H, S, D = q.shape
    L = compressed_k.shape[2]
    SQ = 512
    grid = (H, S // SQ)
    return pl.pallas_call(
        functools.partial(_attn_block_kernel, sq=SQ, L=L,
                          scale=1.0 / math.sqrt(D)),
        grid=grid,
        in_specs=[
            pl.BlockSpec((1, 1, SQ, D), lambda h, sb: (0, h, sb, 0)),
            pl.BlockSpec((1, 1, L, D), lambda h, sb: (0, h, 0, 0)),
            pl.BlockSpec((1, 1, L, D), lambda h, sb: (0, h, 0, 0)),
        ],
        out_specs=pl.BlockSpec((1, 1, SQ, D), lambda h, sb: (0, h, sb, 0)),
        out_shape=jax.ShapeDtypeStruct((B, H, S, D), jnp.float32),
        compiler_params=pltpu.CompilerParams(
            dimension_semantics=("parallel", "parallel")),
    )(q, compressed_k, compressed_v)


# speedup vs baseline: 2.9862x; 1.1367x over previous
import functools
import math

import jax
import jax.numpy as jnp
from jax.experimental import pallas as pl
from jax.experimental.pallas import tpu as pltpu

TOPK = 16
NEG = -1e30


def _attn_block_kernel(q_ref, k_ref, v_ref, o_ref, *, sq, L, scale):
    sb = pl.program_id(1)
    q = q_ref[0, 0] * scale
    k = k_ref[0, 0]
    v = v_ref[0, 0]

    s = jax.lax.dot_general(
        q, k, (((1,), (1,)), ((), ())),
        preferred_element_type=jnp.float32)

    row = jax.lax.broadcasted_iota(jnp.int32, (sq, L), 0) + sb * sq
    col = jax.lax.broadcasted_iota(jnp.int32, (sq, L), 1)
    s = jnp.where(col <= row, s, NEG)

    a, b, c, d = (s[:, 0:128], s[:, 128:256], s[:, 256:384], s[:, 384:512])
    hi1, lo1 = jnp.maximum(a, b), jnp.minimum(a, b)
    hi2, lo2 = jnp.maximum(c, d), jnp.minimum(c, d)
    s1 = jnp.maximum(hi1, hi2)
    th = jnp.minimum(hi1, hi2)
    tl = jnp.maximum(lo1, lo2)
    s4 = jnp.minimum(lo1, lo2)
    s2 = jnp.maximum(th, tl)
    s3 = jnp.minimum(th, tl)

    w = s1
    m = None
    t = None
    for p in range(TOPK):
        t = jnp.max(w, axis=1, keepdims=True)
        if p == 0:
            m = t
        if p < TOPK - 1:
            hit = w == t
            w = jnp.where(hit, s2, w)
            s2 = jnp.where(hit, s3, s2)
            s3 = jnp.where(hit, s4, s3)
            s4 = jnp.where(hit, NEG, s4)

    p_num = jnp.where(s >= t, jnp.exp(s - m), 0.0)
    z = jnp.sum(p_num, axis=1, keepdims=True)
    o = jax.lax.dot_general(
        p_num, v, (((1,), (0,)), ((), ())),
        preferred_element_type=jnp.float32)
    o_ref[0, 0] = o / z


@jax.jit
def kernel(q, compressed_k, compressed_v):
    B, H, S, D = q.shape
    L = compressed_k.shape[2]
    SQ = 512
    grid = (H, S // SQ)
    return pl.pallas_call(
        functools.partial(_attn_block_kernel, sq=SQ, L=L,
                          scale=1.0 / math.sqrt(D)),
        grid=grid,
        in_specs=[
            pl.BlockSpec((1, 1, SQ, D), lambda h, sb: (0, h, sb, 0)),
            pl.BlockSpec((1, 1, L, D), lambda h, sb: (0, h, 0, 0)),
            pl.BlockSpec((1, 1, L, D), lambda h, sb: (0, h, 0, 0)),
        ],
        out_specs=pl.BlockSpec((1, 1, SQ, D), lambda h, sb: (0, h, sb, 0)),
        out_shape=jax.ShapeDtypeStruct((B, H, S, D), jnp.float32),
    )(q, compressed_k, compressed_v)
